# Initial kernel scaffold; baseline (speedup 1.0000x reference)
import jax
import jax.numpy as jnp
from jax.experimental import pallas as pl


def kernel(user_embedding, item_embedding, rows, cols, vals):
    raise NotImplementedError("write your pallas kernel here")


# speedup vs baseline: 5.7718x; 5.7718x over previous
import jax
import jax.numpy as jnp
from jax import lax
from jax.experimental import pallas as pl
from jax.experimental.pallas import tpu as pltpu
from jax.experimental.pallas import tpu_sc as plsc

_L = 16
_K = 128
_NS = 16
_NC = 2
_NLAYER = 3


def _ceil_to(x, m):
    return ((x + m - 1) // m) * m


def _make_layer(hp, d, nch):
    f32, i32 = jnp.float32, jnp.int32
    r_pt = hp // _NS
    nfull, nrem = divmod(r_pt, _K)

    def body(feat, tot_in, epack, evals, feat_nxt, tot_out,
             ebuf, vbuf, gat_v, tot_v, acc_sh, sem):
        c = lax.axis_index("c")
        s = lax.axis_index("s")

        zeros16 = jnp.zeros((_L,), f32)

        def _zrow(i, carry):
            for dd in range(d // _L):
                gat_v[i, pl.ds(dd * _L, _L)] = zeros16
            return carry

        lax.fori_loop(0, _K, _zrow, 0)
        rb = s * r_pt
        for j in range(nfull):
            pltpu.sync_copy(gat_v, acc_sh.at[pl.ds(rb + j * _K, _K)])
        if nrem:
            pltpu.sync_copy(gat_v.at[pl.ds(0, nrem)],
                            acc_sh.at[pl.ds(rb + nfull * _K, nrem)])
        plsc.subcore_barrier()

        bcast_dnums = lax.GatherDimensionNumbers(
            offset_dims=(), collapsed_slice_dims=(0,), start_index_map=(0,))

        def _chunk(ch, carry):
            pltpu.sync_copy(epack.at[c, s, ch], ebuf)
            pltpu.sync_copy(evals.at[c, s, ch], vbuf)
            pltpu.async_copy(feat.at[ebuf.at[0]], gat_v, sem).wait()
            for g in range(_K // _L):
                v16 = vbuf[pl.ds(g * _L, _L)]
                for e in range(_L):
                    vb = lax.gather(
                        v16, jnp.full((_L, 1), e, i32), bcast_dnums, (1,),
                        mode=lax.GatherScatterMode.PROMISE_IN_BOUNDS)
                    row = g * _L + e
                    for dd in range(d // _L):
                        sl = pl.ds(dd * _L, _L)
                        gat_v[row, sl] = gat_v[row, sl] * vb
            pltpu.sync_copy(gat_v, acc_sh.at[ebuf.at[1]], add=True)
            return carry

        lax.fori_loop(0, nch, _chunk, 0)
        plsc.subcore_barrier()

        grb = c * hp + rb
        for j in range(nfull + (1 if nrem else 0)):
            nr = _K if j < nfull else nrem
            pltpu.sync_copy(acc_sh.at[pl.ds(rb + j * _K, nr)],
                            gat_v.at[pl.ds(0, nr)])
            pltpu.sync_copy(tot_in.at[pl.ds(grb + j * _K, nr)],
                            tot_v.at[pl.ds(0, nr)])

            def _add(i, carry):
                for dd in range(d // _L):
                    sl = pl.ds(dd * _L, _L)
                    tot_v[i, sl] = tot_v[i, sl] + gat_v[i, sl]
                return carry

            lax.fori_loop(0, nr, _add, 0)
            pltpu.sync_copy(gat_v.at[pl.ds(0, nr)],
                            feat_nxt.at[pl.ds(grb + j * _K, nr)])
            pltpu.sync_copy(tot_v.at[pl.ds(0, nr)],
                            tot_out.at[pl.ds(grb + j * _K, nr)])

    return pl.kernel(
        body,
        out_type=[jax.ShapeDtypeStruct((2 * hp, d), f32),
                  jax.ShapeDtypeStruct((2 * hp, d), f32)],
        mesh=plsc.VectorSubcoreMesh(core_axis_name="c", subcore_axis_name="s"),
        compiler_params=pltpu.CompilerParams(use_tc_tiling_on_sc=False),
        scratch_types=[
            pltpu.VMEM((2, _K), i32),
            pltpu.VMEM((_K,), f32),
            pltpu.VMEM((_K, d), f32),
            pltpu.VMEM((_K, d), f32),
            pltpu.VMEM_SHARED((hp, d), f32),
            pltpu.SemaphoreType.DMA,
        ],
    )


def kernel(user_embedding, item_embedding, rows, cols, vals):
    f32, i32 = jnp.float32, jnp.int32
    nu, d = user_embedding.shape
    ni = item_embedding.shape[0]
    nnz2 = rows.shape[0] // 2
    hp = _ceil_to(max(nu, ni), _NS * 8)
    t_pt = _ceil_to(-(-nnz2 // _NS), _K)
    nch = t_pt // _K
    pad = _NS * t_pt - nnz2

    feat0 = jnp.zeros((2 * hp, d), f32)
    feat0 = lax.dynamic_update_slice(feat0, user_embedding, (0, 0))
    feat0 = lax.dynamic_update_slice(feat0, item_embedding, (hp, 0))

    colsg = jnp.where(cols >= nu, cols + (hp - nu), cols).astype(i32)
    rowsl = jnp.where(rows >= nu, rows - nu, rows).astype(i32)
    zi = jnp.zeros((pad,), i32)
    zf = jnp.zeros((pad,), f32)

    def seg(x, z):
        return jnp.stack([jnp.concatenate([x[:nnz2], z]),
                          jnp.concatenate([x[nnz2:], z])])

    epack = jnp.stack([seg(colsg, zi), seg(rowsl, zi)], axis=1)
    epack = epack.reshape(_NC, 2, _NS, nch, _K).transpose(0, 2, 3, 1, 4)
    evals = seg(vals.astype(f32), zf).reshape(_NC, _NS, nch, _K)

    layer = _make_layer(hp, d, nch)
    feat, tot = feat0, feat0
    for _ in range(_NLAYER):
        feat, tot = layer(feat, tot, epack, evals)
    return tot[:nu], tot[hp:hp + ni]


# speedup vs baseline: 10.5247x; 1.8235x over previous
import jax
import jax.numpy as jnp
from jax import lax
from jax.experimental import pallas as pl
from jax.experimental.pallas import tpu as pltpu
from jax.experimental.pallas import tpu_sc as plsc

_L = 16
_K = 128
_NS = 16
_NC = 2
_NLAYER = 3


def _ceil_to(x, m):
    return ((x + m - 1) // m) * m


def _make_layer(hp, d, nch):
    f32, i32 = jnp.float32, jnp.int32
    r_pt = hp // _NS
    nfull, nrem = divmod(r_pt, _K)
    assert nch % 2 == 0

    def body(feat, tot_in, ecol, erow, evals, feat_nxt, tot_out,
             cbuf0, cbuf1, rbuf0, rbuf1, vbuf0, vbuf1, gat0, gat1, tot_v,
             acc_sh, sem_c0, sem_c1, sem_r0, sem_r1, sem_g0, sem_g1, sem_s):
        c = lax.axis_index("c")
        s = lax.axis_index("s")
        cbuf = (cbuf0, cbuf1)
        rbuf = (rbuf0, rbuf1)
        vbuf = (vbuf0, vbuf1)
        gat = (gat0, gat1)
        sem_c = (sem_c0, sem_c1)
        sem_r = (sem_r0, sem_r1)
        sem_g = (sem_g0, sem_g1)

        zeros16 = jnp.zeros((_L,), f32)

        def _zrow(i, carry):
            for dd in range(d // _L):
                gat0[i, pl.ds(dd * _L, _L)] = zeros16
            return carry

        lax.fori_loop(0, _K, _zrow, 0)
        rb = s * r_pt
        for j in range(nfull):
            pltpu.sync_copy(gat0, acc_sh.at[pl.ds(rb + j * _K, _K)])
        if nrem:
            pltpu.sync_copy(gat0.at[pl.ds(0, nrem)],
                            acc_sh.at[pl.ds(rb + nfull * _K, nrem)])
        plsc.subcore_barrier()

        def issue_cols(x, b):
            pltpu.async_copy(ecol.at[c, s, x], cbuf[b], sem_c[b])
            pltpu.async_copy(evals.at[c, s, x], vbuf[b], sem_c[b])

        def wait_cols(x, b):
            pltpu.make_async_copy(ecol.at[c, s, x], cbuf[b], sem_c[b]).wait()
            pltpu.make_async_copy(evals.at[c, s, x], vbuf[b], sem_c[b]).wait()

        def issue_rows(x, b):
            pltpu.async_copy(erow.at[c, s, x], rbuf[b], sem_r[b])

        def wait_rows(x, b):
            pltpu.make_async_copy(erow.at[c, s, x], rbuf[b], sem_r[b]).wait()

        def issue_gather(b):
            pltpu.async_copy(feat.at[cbuf[b]], gat[b], sem_g[b])

        def wait_gather(b):
            pltpu.make_async_copy(feat.at[cbuf[b]], gat[b], sem_g[b]).wait()

        def issue_scatter(b):
            pltpu.async_copy(gat[b], acc_sh.at[rbuf[b]], sem_s, add=True)

        def wait_scatter(b):
            pltpu.make_async_copy(gat[b], acc_sh.at[rbuf[b]], sem_s).wait()

        bcast_dnums = lax.GatherDimensionNumbers(
            offset_dims=(), collapsed_slice_dims=(0,), start_index_map=(0,))

        def compute(b):
            gv = gat[b]
            for g in range(_K // _L):
                v16 = vbuf[b][pl.ds(g * _L, _L)]
                for e in range(_L):
                    vb = lax.gather(
                        v16, jnp.full((_L, 1), e, i32), bcast_dnums, (1,),
                        mode=lax.GatherScatterMode.PROMISE_IN_BOUNDS)
                    row = g * _L + e
                    for dd in range(d // _L):
                        sl = pl.ds(dd * _L, _L)
                        gv[row, sl] = gv[row, sl] * vb

        issue_cols(0, 0)
        issue_rows(0, 0)
        wait_cols(0, 0)
        issue_gather(0)
        issue_cols(1, 1)

        def pair(i, carry):
            base = 2 * i

            wait_gather(0)

            @pl.when(base > 0)
            def _():
                wait_scatter(1)
            wait_cols(base + 1, 1)
            issue_gather(1)
            issue_rows(base + 1, 1)
            compute(0)
            wait_rows(base, 0)
            issue_scatter(0)

            @pl.when(base + 2 < nch)
            def _():
                issue_cols(base + 2, 0)

            wait_gather(1)
            wait_scatter(0)

            @pl.when(base + 2 < nch)
            def _():
                wait_cols(base + 2, 0)
                issue_gather(0)
                issue_rows(base + 2, 0)
            compute(1)
            wait_rows(base + 1, 1)
            issue_scatter(1)

            @pl.when(base + 3 < nch)
            def _():
                issue_cols(base + 3, 1)
            return carry

        lax.fori_loop(0, nch // 2, pair, 0)
        wait_scatter(1)
        plsc.subcore_barrier()

        grb = c * hp + rb
        for j in range(nfull + (1 if nrem else 0)):
            nr = _K if j < nfull else nrem
            pltpu.sync_copy(acc_sh.at[pl.ds(rb + j * _K, nr)],
                            gat0.at[pl.ds(0, nr)])
            pltpu.sync_copy(tot_in.at[pl.ds(grb + j * _K, nr)],
                            tot_v.at[pl.ds(0, nr)])

            def _add(i, carry):
                for dd in range(d // _L):
                    sl = pl.ds(dd * _L, _L)
                    tot_v[i, sl] = tot_v[i, sl] + gat0[i, sl]
                return carry

            lax.fori_loop(0, nr, _add, 0)
            pltpu.sync_copy(gat0.at[pl.ds(0, nr)],
                            feat_nxt.at[pl.ds(grb + j * _K, nr)])
            pltpu.sync_copy(tot_v.at[pl.ds(0, nr)],
                            tot_out.at[pl.ds(grb + j * _K, nr)])

    return pl.kernel(
        body,
        out_type=[jax.ShapeDtypeStruct((2 * hp, d), f32),
                  jax.ShapeDtypeStruct((2 * hp, d), f32)],
        mesh=plsc.VectorSubcoreMesh(core_axis_name="c", subcore_axis_name="s"),
        compiler_params=pltpu.CompilerParams(use_tc_tiling_on_sc=False),
        scratch_types=[
            pltpu.VMEM((_K,), i32),
            pltpu.VMEM((_K,), i32),
            pltpu.VMEM((_K,), i32),
            pltpu.VMEM((_K,), i32),
            pltpu.VMEM((_K,), f32),
            pltpu.VMEM((_K,), f32),
            pltpu.VMEM((_K, d), f32),
            pltpu.VMEM((_K, d), f32),
            pltpu.VMEM((_K, d), f32),
            pltpu.VMEM_SHARED((hp, d), f32),
            pltpu.SemaphoreType.DMA,
            pltpu.SemaphoreType.DMA,
            pltpu.SemaphoreType.DMA,
            pltpu.SemaphoreType.DMA,
            pltpu.SemaphoreType.DMA,
            pltpu.SemaphoreType.DMA,
            pltpu.SemaphoreType.DMA,
        ],
    )


def kernel(user_embedding, item_embedding, rows, cols, vals):
    f32, i32 = jnp.float32, jnp.int32
    nu, d = user_embedding.shape
    ni = item_embedding.shape[0]
    nnz2 = rows.shape[0] // 2
    hp = _ceil_to(max(nu, ni), _NS * 8)
    t_pt = _ceil_to(-(-nnz2 // _NS), 2 * _K)
    nch = t_pt // _K
    pad = _NS * t_pt - nnz2

    feat0 = jnp.zeros((2 * hp, d), f32)
    feat0 = lax.dynamic_update_slice(feat0, user_embedding, (0, 0))
    feat0 = lax.dynamic_update_slice(feat0, item_embedding, (hp, 0))

    colsg = jnp.where(cols >= nu, cols + (hp - nu), cols).astype(i32)
    rowsl = jnp.where(rows >= nu, rows - nu, rows).astype(i32)
    zi = jnp.zeros((pad,), i32)
    zf = jnp.zeros((pad,), f32)

    def seg(x, z):
        return jnp.stack([jnp.concatenate([x[:nnz2], z]),
                          jnp.concatenate([x[nnz2:], z])]).reshape(
                              _NC, _NS, nch, _K)

    ecol = seg(colsg, zi)
    erow = seg(rowsl, zi)
    evals = seg(vals.astype(f32), zf)

    layer = _make_layer(hp, d, nch)
    feat, tot = feat0, feat0
    for _ in range(_NLAYER):
        feat, tot = layer(feat, tot, ecol, erow, evals)
    return tot[:nu], tot[hp:hp + ni]
